# Initial kernel scaffold; baseline (speedup 1.0000x reference)
import jax
import jax.numpy as jnp
from jax.experimental import pallas as pl


def kernel(pred_pc, gt_pc):
    raise NotImplementedError("write your pallas kernel here")


# speedup vs baseline: 1.0291x; 1.0291x over previous
import jax
import jax.numpy as jnp
from jax.experimental import pallas as pl
from jax.experimental.pallas import tpu as pltpu

_B, _N, _D = 4, 4096, 3
_TN = 512
_NBLK = _N // _TN


def _chamfer_body(x_ref, g_ref, out_ref, colmin_ref):
    b = pl.program_id(0)
    i = pl.program_id(1)

    x = x_ref[0]
    g = g_ref[0]

    xy = jax.lax.dot_general(
        x, g, (((1,), (0,)), ((), ())),
        preferred_element_type=jnp.float32,
    )
    x2 = jnp.sum(x * x, axis=1, keepdims=True)
    y2 = jnp.sum(g * g, axis=0, keepdims=True)
    d = jnp.maximum(x2 + y2 - 2.0 * xy, 0.0)

    rowmin = jnp.min(d, axis=1)
    colmin = jnp.min(d, axis=0, keepdims=True)[None]

    @pl.when(jnp.logical_and(b == 0, i == 0))
    def _():
        out_ref[...] = jnp.zeros((1, 1), jnp.float32)

    @pl.when(i == 0)
    def _():
        colmin_ref[...] = colmin

    @pl.when(i > 0)
    def _():
        colmin_ref[...] = jnp.minimum(colmin_ref[...], colmin)

    out_ref[...] += jnp.sum(rowmin).reshape(1, 1) * (1.0 / (_B * _N))

    @pl.when(i == _NBLK - 1)
    def _():
        out_ref[...] += jnp.sum(colmin_ref[...]).reshape(1, 1) * (1.0 / (_B * _N))


def kernel(pred_pc, gt_pc):
    gt_t = jnp.transpose(gt_pc, (0, 2, 1))

    out, _ = pl.pallas_call(
        _chamfer_body,
        grid=(_B, _NBLK),
        in_specs=[
            pl.BlockSpec((1, _TN, _D), lambda b, i: (b, i, 0)),
            pl.BlockSpec((1, _D, _N), lambda b, i: (b, 0, 0)),
        ],
        out_specs=[
            pl.BlockSpec((1, 1), lambda b, i: (0, 0)),
            pl.BlockSpec((1, 1, _N), lambda b, i: (b, 0, 0)),
        ],
        out_shape=[
            jax.ShapeDtypeStruct((1, 1), jnp.float32),
            jax.ShapeDtypeStruct((_B, 1, _N), jnp.float32),
        ],
        compiler_params=pltpu.CompilerParams(
            dimension_semantics=("arbitrary", "arbitrary"),
        ),
    )(pred_pc, gt_t)

    return out[0, 0]


# speedup vs baseline: 1.2765x; 1.2404x over previous
import jax
import jax.numpy as jnp
from jax.experimental import pallas as pl
from jax.experimental.pallas import tpu as pltpu

_B, _N, _D = 4, 4096, 3
_TN = 512
_NBLK = _N // _TN


def _chamfer_body(x_ref, g_ref, out_ref, colmin_ref):
    b = pl.program_id(0)
    i = pl.program_id(1)

    x = x_ref[0]
    g = g_ref[0]

    xym2 = jax.lax.dot_general(
        x * (-2.0), g, (((1,), (0,)), ((), ())),
        preferred_element_type=jnp.float32,
    )
    x2 = jnp.sum(x * x, axis=1, keepdims=True)
    y2 = jnp.sum(g * g, axis=0, keepdims=True)
    d = (xym2 + y2) + x2

    rowmin = jnp.maximum(jnp.min(d, axis=1), 0.0)
    colmin = jnp.min(d, axis=0, keepdims=True)[None]

    @pl.when(jnp.logical_and(b == 0, i == 0))
    def _():
        out_ref[...] = jnp.zeros((1, 1), jnp.float32)

    @pl.when(i == 0)
    def _():
        colmin_ref[...] = colmin

    @pl.when(i > 0)
    def _():
        colmin_ref[...] = jnp.minimum(colmin_ref[...], colmin)

    out_ref[...] += jnp.sum(rowmin).reshape(1, 1) * (1.0 / (_B * _N))

    @pl.when(i == _NBLK - 1)
    def _():
        cm = jnp.maximum(colmin_ref[...], 0.0)
        out_ref[...] += jnp.sum(cm).reshape(1, 1) * (1.0 / (_B * _N))


def kernel(pred_pc, gt_pc):
    gt_t = jnp.transpose(gt_pc, (0, 2, 1))

    out, _ = pl.pallas_call(
        _chamfer_body,
        grid=(_B, _NBLK),
        in_specs=[
            pl.BlockSpec((1, _TN, _D), lambda b, i: (b, i, 0)),
            pl.BlockSpec((1, _D, _N), lambda b, i: (b, 0, 0)),
        ],
        out_specs=[
            pl.BlockSpec((1, 1), lambda b, i: (0, 0)),
            pl.BlockSpec((1, 1, _N), lambda b, i: (b, 0, 0)),
        ],
        out_shape=[
            jax.ShapeDtypeStruct((1, 1), jnp.float32),
            jax.ShapeDtypeStruct((_B, 1, _N), jnp.float32),
        ],
        compiler_params=pltpu.CompilerParams(
            dimension_semantics=("arbitrary", "arbitrary"),
        ),
    )(pred_pc, gt_t)

    return out[0, 0]


# speedup vs baseline: 1.3619x; 1.0669x over previous
import jax
import jax.numpy as jnp
from jax.experimental import pallas as pl
from jax.experimental.pallas import tpu as pltpu

_B, _N, _D = 4, 4096, 3
_TN = 512
_NBLK = _N // _TN


def _chamfer_body(x_ref, g_ref, out_ref, colmin_ref):
    b = pl.program_id(0)
    i = pl.program_id(1)

    x = x_ref[0]
    g = g_ref[0]

    x2 = jnp.sum(x * x, axis=1, keepdims=True)
    y2 = jnp.sum(g * g, axis=0, keepdims=True)

    ax = (x2.astype(jnp.bfloat16)).astype(jnp.float32)
    bx = x2 - ax
    ay = (y2.astype(jnp.bfloat16)).astype(jnp.float32)
    by = y2 - ay
    ones_x = jnp.ones((x.shape[0], 1), jnp.float32)
    ones_g = jnp.ones((1, g.shape[1]), jnp.float32)
    x_aug = jnp.concatenate([x * (-2.0), ones_x, ones_x, ax, bx], axis=1)
    g_aug = jnp.concatenate([g, ay, by, ones_g, ones_g], axis=0)
    d = jax.lax.dot_general(
        x_aug, g_aug, (((1,), (0,)), ((), ())),
        preferred_element_type=jnp.float32,
    )

    rowmin = jnp.maximum(jnp.min(d, axis=1), 0.0)
    colmin = jnp.min(d, axis=0, keepdims=True)[None]

    @pl.when(jnp.logical_and(b == 0, i == 0))
    def _():
        out_ref[...] = jnp.zeros((1, 1), jnp.float32)

    @pl.when(i == 0)
    def _():
        colmin_ref[...] = colmin

    @pl.when(i > 0)
    def _():
        colmin_ref[...] = jnp.minimum(colmin_ref[...], colmin)

    out_ref[...] += jnp.sum(rowmin).reshape(1, 1) * (1.0 / (_B * _N))

    @pl.when(i == _NBLK - 1)
    def _():
        cm = jnp.maximum(colmin_ref[...], 0.0)
        out_ref[...] += jnp.sum(cm).reshape(1, 1) * (1.0 / (_B * _N))


def kernel(pred_pc, gt_pc):
    gt_t = jnp.transpose(gt_pc, (0, 2, 1))

    out, _ = pl.pallas_call(
        _chamfer_body,
        grid=(_B, _NBLK),
        in_specs=[
            pl.BlockSpec((1, _TN, _D), lambda b, i: (b, i, 0)),
            pl.BlockSpec((1, _D, _N), lambda b, i: (b, 0, 0)),
        ],
        out_specs=[
            pl.BlockSpec((1, 1), lambda b, i: (0, 0)),
            pl.BlockSpec((1, 1, _N), lambda b, i: (b, 0, 0)),
        ],
        out_shape=[
            jax.ShapeDtypeStruct((1, 1), jnp.float32),
            jax.ShapeDtypeStruct((_B, 1, _N), jnp.float32),
        ],
        compiler_params=pltpu.CompilerParams(
            dimension_semantics=("arbitrary", "arbitrary"),
        ),
    )(pred_pc, gt_t)

    return out[0, 0]


# speedup vs baseline: 1.3638x; 1.0014x over previous
import jax
import jax.numpy as jnp
from jax.experimental import pallas as pl
from jax.experimental.pallas import tpu as pltpu

_B, _N, _D = 4, 4096, 3
_TN = 512
_NBLK = _N // _TN


def _chamfer_body(x_ref, g_ref, out_ref, colmin_ref):
    b = pl.program_id(0)
    i = pl.program_id(1)

    x = x_ref[0]
    g = g_ref[0]

    x2 = jnp.sum(x * x, axis=1, keepdims=True)
    y2 = jnp.sum(g * g, axis=0, keepdims=True)

    ax = (x2.astype(jnp.bfloat16)).astype(jnp.float32)
    bx = x2 - ax
    ay = (y2.astype(jnp.bfloat16)).astype(jnp.float32)
    by = y2 - ay
    ones_x = jnp.ones((x.shape[0], 1), jnp.float32)
    ones_g = jnp.ones((1, g.shape[1]), jnp.float32)
    x_aug = jnp.concatenate([x * (-2.0), ones_x, ones_x, ax, bx], axis=1)
    g_aug = jnp.concatenate([g, ay, by, ones_g, ones_g], axis=0)
    _CHUNKS = 4
    _CN = _N // _CHUNKS
    rowmin = None
    colmin_parts = []
    for c in range(_CHUNKS):
        gc = g_aug[:, c * _CN:(c + 1) * _CN]
        dc = jax.lax.dot_general(
            x_aug, gc, (((1,), (0,)), ((), ())),
            preferred_element_type=jnp.float32,
        )
        rm = jnp.min(dc, axis=1)
        rowmin = rm if rowmin is None else jnp.minimum(rowmin, rm)
        colmin_parts.append(jnp.min(dc, axis=0, keepdims=True))

    rowmin = jnp.maximum(rowmin, 0.0)
    colmin = jnp.concatenate(colmin_parts, axis=1)[None]

    @pl.when(jnp.logical_and(b == 0, i == 0))
    def _():
        out_ref[...] = jnp.zeros((1, 1), jnp.float32)

    @pl.when(i == 0)
    def _():
        colmin_ref[...] = colmin

    @pl.when(i > 0)
    def _():
        colmin_ref[...] = jnp.minimum(colmin_ref[...], colmin)

    out_ref[...] += jnp.sum(rowmin).reshape(1, 1) * (1.0 / (_B * _N))

    @pl.when(i == _NBLK - 1)
    def _():
        cm = jnp.maximum(colmin_ref[...], 0.0)
        out_ref[...] += jnp.sum(cm).reshape(1, 1) * (1.0 / (_B * _N))


def kernel(pred_pc, gt_pc):
    gt_t = jnp.transpose(gt_pc, (0, 2, 1))

    out, _ = pl.pallas_call(
        _chamfer_body,
        grid=(_B, _NBLK),
        in_specs=[
            pl.BlockSpec((1, _TN, _D), lambda b, i: (b, i, 0)),
            pl.BlockSpec((1, _D, _N), lambda b, i: (b, 0, 0)),
        ],
        out_specs=[
            pl.BlockSpec((1, 1), lambda b, i: (0, 0)),
            pl.BlockSpec((1, 1, _N), lambda b, i: (b, 0, 0)),
        ],
        out_shape=[
            jax.ShapeDtypeStruct((1, 1), jnp.float32),
            jax.ShapeDtypeStruct((_B, 1, _N), jnp.float32),
        ],
        compiler_params=pltpu.CompilerParams(
            dimension_semantics=("arbitrary", "arbitrary"),
        ),
    )(pred_pc, gt_t)

    return out[0, 0]
